# baseline (speedup 1.0000x reference)
import jax
import jax.numpy as jnp
from jax import lax
from jax.experimental import pallas as pl
from jax.experimental.pallas import tpu as pltpu
from jax.experimental.pallas import tpu_sc as plsc

N = 10000
NP = 10240
E = 160000
EPAD = 163840
D_IN = 256
D_OUT = 256
H = 128
NC = 2
NS = 16
LANES = 16
EW = 128
EROWS = EPAD // EW
ROWS_PER_TILE = EROWS // NS
HALFR = ROWS_PER_TILE // 2
ROWS_PER_TILE_A = EROWS // (NC * NS)
NPT = NP // NS
BM = 2048


def _deg_body(dst_hbm, deg_out, acc, idx, ones, zbuf, ssem):
    c = lax.axis_index("c")
    s = lax.axis_index("s")
    nbase = s * NPT

    onev = jnp.ones((LANES,), jnp.float32)
    for j in range(EW // LANES):
        ones[pl.ds(j * LANES, LANES)] = onev
    zv = jnp.zeros((LANES,), jnp.float32)

    def fill_z(i, carry):
        zbuf[pl.ds(i * LANES, LANES)] = zv
        return carry
    lax.fori_loop(0, NPT // LANES, fill_z, 0)

    abase = (c * NS + s) * ROWS_PER_TILE_A
    pltpu.sync_copy(dst_hbm.at[pl.ds(abase, ROWS_PER_TILE_A)], idx)
    pltpu.sync_copy(zbuf, acc.at[pl.ds(nbase, NPT)])
    plsc.subcore_barrier()

    for j in range(ROWS_PER_TILE_A):
        pltpu.async_copy(ones, acc.at[idx.at[j]], ssem,
                         add=True)
    for j in range(ROWS_PER_TILE_A):
        pltpu.make_async_copy(ones, acc.at[idx.at[j]],
                              ssem).wait()

    plsc.subcore_barrier()
    pltpu.sync_copy(acc.at[pl.ds(nbase, NPT)],
                    deg_out.at[pl.ds(c * NP + nbase, NPT)])


def _agg_body(h_hbm, src_hbm, dst_hbm, tmp_out, acc, isrc, idst, rows,
              gs0, gs1, ism):
    c = lax.axis_index("c")
    s = lax.axis_index("s")
    nbase = s * NPT
    ebase = s * ROWS_PER_TILE

    ini = pltpu.async_copy(h_hbm.at[c, pl.ds(nbase, NPT)],
                           acc.at[pl.ds(nbase, NPT)], ism)
    pltpu.sync_copy(src_hbm.at[pl.ds(ebase, ROWS_PER_TILE)], isrc)
    pltpu.sync_copy(dst_hbm.at[pl.ds(ebase, HALFR)], idst)
    pltpu.async_copy(h_hbm.at[c].at[isrc.at[0]], rows.at[0], gs0)
    ini.wait()
    plsc.subcore_barrier()

    for half in range(2):
        roff = half * HALFR

        def step(k, carry):
            r0 = roff + 2 * k
            pltpu.async_copy(h_hbm.at[c].at[isrc.at[r0 + 1]],
                             rows.at[1], gs1)
            pltpu.make_async_copy(h_hbm.at[c].at[isrc.at[r0]],
                                  rows.at[0], gs0).wait()
            pltpu.sync_copy(rows.at[0], acc.at[idst.at[2 * k]], add=True)

            if half == 0:
                pltpu.async_copy(h_hbm.at[c].at[isrc.at[r0 + 2]],
                                 rows.at[0], gs0)
            else:
                @pl.when(k < HALFR // 2 - 1)
                def _():
                    pltpu.async_copy(h_hbm.at[c].at[isrc.at[r0 + 2]],
                                     rows.at[0], gs0)

            pltpu.make_async_copy(h_hbm.at[c].at[isrc.at[r0 + 1]],
                                  rows.at[1], gs1).wait()
            pltpu.sync_copy(rows.at[1], acc.at[idst.at[2 * k + 1]],
                            add=True)
            return carry
        lax.fori_loop(0, HALFR // 2, step, 0)
        if half == 0:
            pltpu.sync_copy(dst_hbm.at[pl.ds(ebase + HALFR, HALFR)], idst)

    plsc.subcore_barrier()
    pltpu.sync_copy(acc.at[pl.ds(nbase, NPT)],
                    tmp_out.at[c, pl.ds(nbase, NPT)])


def _mm_body(x_ref, w_ref, dl_ref, dh_ref, out_ref):
    deg = 1.0 + (dl_ref[...] + dh_ref[...]).reshape(BM, 1)
    dinv = lax.rsqrt(deg)
    h = jnp.dot(x_ref[...], w_ref[...], preferred_element_type=jnp.float32)
    out_ref[0] = h[:, :H] * dinv
    out_ref[1] = h[:, H:] * dinv


def _post_body(tl_ref, th_ref, dl_ref, dh_ref, b_ref, out_ref):
    deg = 1.0 + (dl_ref[...] + dh_ref[...]).reshape(BM, 1)
    dinv = lax.rsqrt(deg)
    out_ref[...] = (
        jnp.concatenate([tl_ref[0] * dinv, th_ref[0] * dinv], axis=1)
        + b_ref[...]
    )


def kernel(x, edge_index, W, b):
    pad = N + (jnp.arange(EPAD - E, dtype=jnp.int32) % (NP - N))
    ei_b = lax.optimization_barrier(edge_index)
    src2 = jnp.concatenate([ei_b[0], pad]).reshape(EROWS, EW)
    dst2 = jnp.concatenate([edge_index[1], pad]).reshape(EROWS, EW)
    mesh = plsc.VectorSubcoreMesh(core_axis_name="c", subcore_axis_name="s")

    deg_flat = pl.kernel(
        _deg_body,
        out_type=jax.ShapeDtypeStruct((NC * NP,), jnp.float32),
        mesh=mesh,
        scratch_types=[
            pltpu.VMEM_SHARED((NP,), jnp.float32),
            pltpu.VMEM((ROWS_PER_TILE_A, EW), jnp.int32),
            pltpu.VMEM((EW,), jnp.float32),
            pltpu.VMEM((NPT,), jnp.float32),
            pltpu.SemaphoreType.DMA,
        ],
    )(dst2)

    h3 = pl.pallas_call(
        _mm_body,
        grid=(NP // BM,),
        in_specs=[
            pl.BlockSpec((BM, D_IN), lambda i: (i, 0)),
            pl.BlockSpec((D_IN, D_OUT), lambda i: (0, 0)),
            pl.BlockSpec((BM,), lambda i: (i,)),
            pl.BlockSpec((BM,), lambda i: (i + NP // BM,)),
        ],
        out_specs=pl.BlockSpec((2, BM, H), lambda i: (0, i, 0)),
        out_shape=jax.ShapeDtypeStruct((NC, NP, H), jnp.float32),
    )(x, W, deg_flat, deg_flat)

    tmp = pl.kernel(
        _agg_body,
        out_type=jax.ShapeDtypeStruct((NC, NP, H), jnp.float32),
        mesh=mesh,
        scratch_types=[
            pltpu.VMEM_SHARED((NP, H), jnp.float32),
            pltpu.VMEM((ROWS_PER_TILE, EW), jnp.int32),
            pltpu.VMEM((HALFR, EW), jnp.int32),
            pltpu.VMEM((2, EW, H), jnp.float32),
            pltpu.SemaphoreType.DMA,
            pltpu.SemaphoreType.DMA,
            pltpu.SemaphoreType.DMA,
        ],
    )(h3, src2, dst2)

    out = pl.pallas_call(
        _post_body,
        grid=(NP // BM,),
        in_specs=[
            pl.BlockSpec((1, BM, H), lambda i: (0, i, 0)),
            pl.BlockSpec((1, BM, H), lambda i: (1, i, 0)),
            pl.BlockSpec((BM,), lambda i: (i,)),
            pl.BlockSpec((BM,), lambda i: (i + NP // BM,)),
            pl.BlockSpec((1, D_OUT), lambda i: (0, 0)),
        ],
        out_specs=pl.BlockSpec((BM, D_OUT), lambda i: (i, 0)),
        out_shape=jax.ShapeDtypeStruct((N, D_OUT), jnp.float32),
    )(tmp, tmp, deg_flat, deg_flat, b.reshape(1, D_OUT))

    return (out, edge_index)


# speedup vs baseline: 1.0036x; 1.0036x over previous
import jax
import jax.numpy as jnp
from jax import lax
from jax.experimental import pallas as pl
from jax.experimental.pallas import tpu as pltpu
from jax.experimental.pallas import tpu_sc as plsc

N = 10000
NP = 10240
E = 160000
EPAD = 163840
D_IN = 256
D_OUT = 256
H = 128
NC = 2
NS = 16
LANES = 16
EW = 128
EROWS = EPAD // EW
ROWS_PER_TILE = EROWS // NS
HALFR = ROWS_PER_TILE // 2
ROWS_PER_TILE_A = EROWS // (NC * NS)
NPT = NP // NS
BM = 2048


def _deg_body(dst_hbm, deg_out, acc, idx, ones, zbuf, ssem):
    c = lax.axis_index("c")
    s = lax.axis_index("s")
    nbase = s * NPT

    onev = jnp.ones((LANES,), jnp.float32)
    for j in range(EW // LANES):
        ones[pl.ds(j * LANES, LANES)] = onev
    zv = jnp.zeros((LANES,), jnp.float32)

    def fill_z(i, carry):
        zbuf[pl.ds(i * LANES, LANES)] = zv
        return carry
    lax.fori_loop(0, NPT // LANES, fill_z, 0)

    abase = (c * NS + s) * ROWS_PER_TILE_A
    pltpu.sync_copy(dst_hbm.at[pl.ds(abase, ROWS_PER_TILE_A)], idx)
    pltpu.sync_copy(zbuf, acc.at[pl.ds(nbase, NPT)])
    plsc.subcore_barrier()

    for j in range(ROWS_PER_TILE_A):
        pltpu.async_copy(ones, acc.at[idx.at[j]], ssem,
                         add=True)
    for j in range(ROWS_PER_TILE_A):
        pltpu.make_async_copy(ones, acc.at[idx.at[j]],
                              ssem).wait()

    plsc.subcore_barrier()
    pltpu.sync_copy(acc.at[pl.ds(nbase, NPT)],
                    deg_out.at[pl.ds(c * NP + nbase, NPT)])


def _agg_body(h_hbm, src_hbm, dst_hbm, tmp_out, acc, isrc, idst, rows,
              gs0, gs1, ism):
    c = lax.axis_index("c")
    s = lax.axis_index("s")
    nbase = s * NPT
    ebase = s * ROWS_PER_TILE

    ini = pltpu.async_copy(h_hbm.at[c, pl.ds(nbase, NPT)],
                           acc.at[pl.ds(nbase, NPT)], ism)
    pltpu.sync_copy(src_hbm.at[pl.ds(ebase, ROWS_PER_TILE)], isrc)
    pltpu.sync_copy(dst_hbm.at[pl.ds(ebase, HALFR)], idst)
    pltpu.async_copy(h_hbm.at[c].at[isrc.at[0]], rows.at[0], gs0)
    ini.wait()
    plsc.subcore_barrier()

    for half in range(2):
        roff = half * HALFR

        def step(k, carry):
            r0 = roff + 2 * k
            pltpu.async_copy(h_hbm.at[c].at[isrc.at[r0 + 1]],
                             rows.at[1], gs1)
            pltpu.make_async_copy(h_hbm.at[c].at[isrc.at[r0]],
                                  rows.at[0], gs0).wait()
            pltpu.sync_copy(rows.at[0], acc.at[idst.at[2 * k]], add=True)

            if half == 0:
                pltpu.async_copy(h_hbm.at[c].at[isrc.at[r0 + 2]],
                                 rows.at[0], gs0)
            else:
                @pl.when(k < HALFR // 2 - 1)
                def _():
                    pltpu.async_copy(h_hbm.at[c].at[isrc.at[r0 + 2]],
                                     rows.at[0], gs0)

            pltpu.make_async_copy(h_hbm.at[c].at[isrc.at[r0 + 1]],
                                  rows.at[1], gs1).wait()
            pltpu.sync_copy(rows.at[1], acc.at[idst.at[2 * k + 1]],
                            add=True)
            return carry
        lax.fori_loop(0, HALFR // 2, step, 0)
        if half == 0:
            pltpu.sync_copy(dst_hbm.at[pl.ds(ebase + HALFR, HALFR)], idst)

    plsc.subcore_barrier()
    pltpu.sync_copy(acc.at[pl.ds(nbase, NPT)],
                    tmp_out.at[c, pl.ds(nbase, NPT)])


def _mm_body(x_ref, w_ref, dl_ref, dh_ref, out_ref):
    deg = 1.0 + (dl_ref[...] + dh_ref[...]).reshape(BM, 1)
    dinv = lax.rsqrt(deg)
    h = jnp.dot(x_ref[...], w_ref[...], preferred_element_type=jnp.float32)
    out_ref[0] = h[:, :H] * dinv
    out_ref[1] = h[:, H:] * dinv


def _post_body(tl_ref, th_ref, dl_ref, dh_ref, b_ref, out_ref):
    deg = 1.0 + (dl_ref[...] + dh_ref[...]).reshape(BM, 1)
    dinv = lax.rsqrt(deg)
    out_ref[...] = (
        jnp.concatenate([tl_ref[0] * dinv, th_ref[0] * dinv], axis=1)
        + b_ref[...]
    )


def kernel(x, edge_index, W, b):
    pad = N + (jnp.arange(EPAD - E, dtype=jnp.int32) % (NP - N))
    src2 = jnp.concatenate([edge_index[0], pad]).reshape(EROWS, EW)
    dst2 = jnp.concatenate([edge_index[1], pad]).reshape(EROWS, EW)
    mesh = plsc.VectorSubcoreMesh(core_axis_name="c", subcore_axis_name="s")

    deg_flat = pl.kernel(
        _deg_body,
        out_type=jax.ShapeDtypeStruct((NC * NP,), jnp.float32),
        mesh=mesh,
        scratch_types=[
            pltpu.VMEM_SHARED((NP,), jnp.float32),
            pltpu.VMEM((ROWS_PER_TILE_A, EW), jnp.int32),
            pltpu.VMEM((EW,), jnp.float32),
            pltpu.VMEM((NPT,), jnp.float32),
            pltpu.SemaphoreType.DMA,
        ],
    )(dst2)

    h3 = pl.pallas_call(
        _mm_body,
        grid=(NP // BM,),
        in_specs=[
            pl.BlockSpec((BM, D_IN), lambda i: (i, 0)),
            pl.BlockSpec((D_IN, D_OUT), lambda i: (0, 0)),
            pl.BlockSpec((BM,), lambda i: (i,)),
            pl.BlockSpec((BM,), lambda i: (i + NP // BM,)),
        ],
        out_specs=pl.BlockSpec((2, BM, H), lambda i: (0, i, 0)),
        out_shape=jax.ShapeDtypeStruct((NC, NP, H), jnp.float32),
    )(x, W, deg_flat, deg_flat)

    tmp = pl.kernel(
        _agg_body,
        out_type=jax.ShapeDtypeStruct((NC, NP, H), jnp.float32),
        mesh=mesh,
        scratch_types=[
            pltpu.VMEM_SHARED((NP, H), jnp.float32),
            pltpu.VMEM((ROWS_PER_TILE, EW), jnp.int32),
            pltpu.VMEM((HALFR, EW), jnp.int32),
            pltpu.VMEM((2, EW, H), jnp.float32),
            pltpu.SemaphoreType.DMA,
            pltpu.SemaphoreType.DMA,
            pltpu.SemaphoreType.DMA,
        ],
    )(h3, src2, dst2)

    out = pl.pallas_call(
        _post_body,
        grid=(NP // BM,),
        in_specs=[
            pl.BlockSpec((1, BM, H), lambda i: (0, i, 0)),
            pl.BlockSpec((1, BM, H), lambda i: (1, i, 0)),
            pl.BlockSpec((BM,), lambda i: (i,)),
            pl.BlockSpec((BM,), lambda i: (i + NP // BM,)),
            pl.BlockSpec((1, D_OUT), lambda i: (0, 0)),
        ],
        out_specs=pl.BlockSpec((BM, D_OUT), lambda i: (i, 0)),
        out_shape=jax.ShapeDtypeStruct((N, D_OUT), jnp.float32),
    )(tmp, tmp, deg_flat, deg_flat, b.reshape(1, D_OUT))

    return (out, edge_index)


# speedup vs baseline: 1.0240x; 1.0203x over previous
import jax
import jax.numpy as jnp
from jax import lax
from jax.experimental import pallas as pl
from jax.experimental.pallas import tpu as pltpu
from jax.experimental.pallas import tpu_sc as plsc

N = 10000
NP = 10240
E = 160000
D_IN = 256
D_OUT = 256
H = 128
NC = 2
NS = 16
LANES = 16
EW = 128
ECOLS = E // EW
CPT = 78
SPAIR = CPT // 2
CPA = 39
TAIL0 = 1248
NPT = NP // NS
BM = 2048


def _deg_body(e_hbm, deg_out, acc, eidx, dsta, tidx, ones, zbuf, ssem):
    c = lax.axis_index("c")
    s = lax.axis_index("s")
    t = c * NS + s
    nbase = s * NPT

    onev = jnp.ones((LANES,), jnp.float32)
    for j in range(EW // LANES):
        ones[pl.ds(j * LANES, LANES)] = onev
    zv = jnp.zeros((LANES,), jnp.float32)

    def fill_z(i, carry):
        zbuf[pl.ds(i * LANES, LANES)] = zv
        return carry
    lax.fori_loop(0, NPT // LANES, fill_z, 0)

    pltpu.sync_copy(e_hbm.at[:, pl.ds(t * CPA * EW, CPA * EW)], eidx)
    pltpu.sync_copy(zbuf, acc.at[pl.ds(nbase, NPT)])

    def stage(r, carry):
        for j in range(EW // LANES):
            dsta[r, pl.ds(j * LANES, LANES)] = (
                eidx[1, pl.ds(r * EW + j * LANES, LANES)])
        return carry
    lax.fori_loop(0, CPA, stage, 0)
    plsc.subcore_barrier()

    for r in range(CPA):
        pltpu.async_copy(ones, acc.at[dsta.at[r]], ssem, add=True)

    @pl.when(t < ECOLS - TAIL0)
    def _():
        pltpu.sync_copy(e_hbm.at[:, pl.ds((TAIL0 + t) * EW, EW)], tidx)
        for j in range(EW // LANES):
            dsta[CPA, pl.ds(j * LANES, LANES)] = (
                tidx[1, pl.ds(j * LANES, LANES)])
        pltpu.sync_copy(ones, acc.at[dsta.at[CPA]], add=True)

    for r in range(CPA):
        pltpu.make_async_copy(ones, acc.at[dsta.at[r]], ssem).wait()

    plsc.subcore_barrier()
    pltpu.sync_copy(acc.at[pl.ds(nbase, NPT)],
                    deg_out.at[pl.ds(c * NP + nbase, NPT)])


def _agg_body(h_hbm, e_hbm, tmp_out, acc, eidx, idst4, tidx, rows,
              gs0, gs1, ism):
    c = lax.axis_index("c")
    s = lax.axis_index("s")
    nbase = s * NPT
    cb = s * CPT

    def stage(slot, row):
        for j in range(EW // LANES):
            idst4[2 * slot, pl.ds(j * LANES, LANES)] = (
                eidx[row, 1, pl.ds(j * LANES, LANES)])
            idst4[2 * slot + 1, pl.ds(j * LANES, LANES)] = (
                eidx[row, 1, pl.ds(EW + j * LANES, LANES)])

    ini = pltpu.async_copy(h_hbm.at[c, pl.ds(nbase, NPT)],
                           acc.at[pl.ds(nbase, NPT)], ism)
    pltpu.sync_copy(e_hbm.at[:, pl.ds(cb * EW, 2 * EW)], eidx.at[0])
    stage(0, 0)
    pltpu.async_copy(h_hbm.at[c].at[eidx.at[0, 0, pl.ds(0, EW)]],
                     rows.at[0], gs0)
    ini.wait()
    plsc.subcore_barrier()

    def step(k, carry):
        p = k % 2
        q = 1 - p

        @pl.when(k < SPAIR - 1)
        def _():
            pltpu.async_copy(e_hbm.at[:, pl.ds((cb + 2 * k + 2) * EW,
                                               2 * EW)],
                             eidx.at[q], ism)

        pltpu.async_copy(h_hbm.at[c].at[eidx.at[p, 0, pl.ds(EW, EW)]],
                         rows.at[1], gs1)
        pltpu.make_async_copy(h_hbm.at[c].at[eidx.at[p, 0, pl.ds(0, EW)]],
                              rows.at[0], gs0).wait()
        pltpu.sync_copy(rows.at[0], acc.at[idst4.at[2 * p]], add=True)

        @pl.when(k < SPAIR - 1)
        def _():
            pltpu.make_async_copy(e_hbm.at[:, pl.ds((cb + 2 * k + 2) * EW,
                                                    2 * EW)],
                                  eidx.at[q], ism).wait()
            stage(q, q)
            pltpu.async_copy(h_hbm.at[c].at[eidx.at[q, 0, pl.ds(0, EW)]],
                             rows.at[0], gs0)

        pltpu.make_async_copy(h_hbm.at[c].at[eidx.at[p, 0, pl.ds(EW, EW)]],
                              rows.at[1], gs1).wait()
        pltpu.sync_copy(rows.at[1], acc.at[idst4.at[2 * p + 1]], add=True)
        return carry
    lax.fori_loop(0, SPAIR, step, 0)

    @pl.when(s < ECOLS - TAIL0)
    def _():
        pltpu.sync_copy(e_hbm.at[:, pl.ds((TAIL0 + s) * EW, EW)], tidx)
        for j in range(EW // LANES):
            idst4[0, pl.ds(j * LANES, LANES)] = (
                tidx[1, pl.ds(j * LANES, LANES)])
        pltpu.async_copy(h_hbm.at[c].at[tidx.at[0]],
                         rows.at[0], gs0).wait()
        pltpu.sync_copy(rows.at[0], acc.at[idst4.at[0]], add=True)

    plsc.subcore_barrier()
    pltpu.sync_copy(acc.at[pl.ds(nbase, NPT)],
                    tmp_out.at[c, pl.ds(nbase, NPT)])


def _mm_body(x_ref, w_ref, dl_ref, dh_ref, out_ref):
    deg = 1.0 + (dl_ref[...] + dh_ref[...]).reshape(BM, 1)
    dinv = lax.rsqrt(deg)
    h = jnp.dot(x_ref[...], w_ref[...], preferred_element_type=jnp.float32)
    out_ref[0] = h[:, :H] * dinv
    out_ref[1] = h[:, H:] * dinv


def _post_body(tl_ref, th_ref, dl_ref, dh_ref, b_ref, out_ref):
    deg = 1.0 + (dl_ref[...] + dh_ref[...]).reshape(BM, 1)
    dinv = lax.rsqrt(deg)
    out_ref[...] = (
        jnp.concatenate([tl_ref[0] * dinv, th_ref[0] * dinv], axis=1)
        + b_ref[...]
    )


def kernel(x, edge_index, W, b):
    mesh = plsc.VectorSubcoreMesh(core_axis_name="c", subcore_axis_name="s")

    deg_flat = pl.kernel(
        _deg_body,
        out_type=jax.ShapeDtypeStruct((NC * NP,), jnp.float32),
        mesh=mesh,
        scratch_types=[
            pltpu.VMEM_SHARED((NP,), jnp.float32),
            pltpu.VMEM((2, CPA * EW), jnp.int32),
            pltpu.VMEM((CPA + 1, EW), jnp.int32),
            pltpu.VMEM((2, EW), jnp.int32),
            pltpu.VMEM((EW,), jnp.float32),
            pltpu.VMEM((NPT,), jnp.float32),
            pltpu.SemaphoreType.DMA,
        ],
    )(edge_index)

    h3 = pl.pallas_call(
        _mm_body,
        grid=(NP // BM,),
        in_specs=[
            pl.BlockSpec((BM, D_IN), lambda i: (i, 0)),
            pl.BlockSpec((D_IN, D_OUT), lambda i: (0, 0)),
            pl.BlockSpec((BM,), lambda i: (i,)),
            pl.BlockSpec((BM,), lambda i: (i + NP // BM,)),
        ],
        out_specs=pl.BlockSpec((2, BM, H), lambda i: (0, i, 0)),
        out_shape=jax.ShapeDtypeStruct((NC, NP, H), jnp.float32),
    )(x, W, deg_flat, deg_flat)

    tmp = pl.kernel(
        _agg_body,
        out_type=jax.ShapeDtypeStruct((NC, NP, H), jnp.float32),
        mesh=mesh,
        scratch_types=[
            pltpu.VMEM_SHARED((NP, H), jnp.float32),
            pltpu.VMEM((2, 2, 2 * EW), jnp.int32),
            pltpu.VMEM((4, EW), jnp.int32),
            pltpu.VMEM((2, EW), jnp.int32),
            pltpu.VMEM((2, EW, H), jnp.float32),
            pltpu.SemaphoreType.DMA,
            pltpu.SemaphoreType.DMA,
            pltpu.SemaphoreType.DMA,
        ],
    )(h3, edge_index)

    out = pl.pallas_call(
        _post_body,
        grid=(NP // BM,),
        in_specs=[
            pl.BlockSpec((1, BM, H), lambda i: (0, i, 0)),
            pl.BlockSpec((1, BM, H), lambda i: (1, i, 0)),
            pl.BlockSpec((BM,), lambda i: (i,)),
            pl.BlockSpec((BM,), lambda i: (i + NP // BM,)),
            pl.BlockSpec((1, D_OUT), lambda i: (0, 0)),
        ],
        out_specs=pl.BlockSpec((BM, D_OUT), lambda i: (i, 0)),
        out_shape=jax.ShapeDtypeStruct((N, D_OUT), jnp.float32),
    )(tmp, tmp, deg_flat, deg_flat, b.reshape(1, D_OUT))

    return (out, edge_index)
